# baseline (speedup 1.0000x reference)
import functools

import jax
import jax.numpy as jnp
from jax import lax
from jax.experimental import pallas as pl
from jax.experimental.pallas import tpu as pltpu
from jax.experimental.pallas import tpu_sc as plsc

NUM_TOKENS = 100000
EMBED_DIM = 300
HEAD = 256
TAIL = EMBED_DIM - HEAD
BATCH = 4096
SEQ = 200

NC = 2
NS = 16
NW = NC * NS

KSPLIT = 4
BATCH_K = BATCH // KSPLIT
CHUNK = 40
CPB = SEQ // CHUNK
N_FLAT = BATCH_K * SEQ
B_PER_W = N_FLAT // NW
BATCH_PER_W = BATCH_K // NW
N_CHUNKS = B_PER_W // CHUNK
ROW_UNROLL = 8


def _sc_gather(table, tail_tab, idx_flat):
    mesh = plsc.VectorSubcoreMesh(core_axis_name="c", subcore_axis_name="s")

    @functools.partial(
        pl.kernel,
        mesh=mesh,
        out_type=jax.ShapeDtypeStruct((BATCH_K, SEQ, EMBED_DIM), jnp.float32),
        scratch_types=[
            pltpu.VMEM((B_PER_W,), jnp.int32),
            pltpu.VMEM((CHUNK, EMBED_DIM), jnp.float32),
            pltpu.VMEM((CHUNK, EMBED_DIM), jnp.float32),
            pltpu.VMEM((CHUNK, 128), jnp.float32),
            pltpu.VMEM((CHUNK, 128), jnp.float32),
            pltpu.SemaphoreType.DMA,
            pltpu.SemaphoreType.DMA,
            pltpu.SemaphoreType.DMA,
            pltpu.SemaphoreType.DMA,
        ],
    )
    def k(table_hbm, tail_hbm, idx_hbm, out_hbm,
          idx_v, rows0, rows1, tail0, tail1, gs0, gs1, ss0, ss1):
        wid = lax.axis_index("s") * NC + lax.axis_index("c")
        base = wid * B_PER_W
        batch0 = wid * BATCH_PER_W
        pltpu.sync_copy(idx_hbm.at[pl.ds(base, B_PER_W)], idx_v)

        rows = (rows0, rows1)
        tails = (tail0, tail1)
        gsems = (gs0, gs1)
        ssems = (ss0, ss1)

        def gather(c, b):
            idx_c = idx_v.at[pl.ds(c * CHUNK, CHUNK)]
            pltpu.make_async_copy(
                table_hbm.at[idx_c, pl.ds(0, HEAD)],
                rows[b].at[:, pl.ds(0, HEAD)],
                gsems[b],
            ).start()
            pltpu.make_async_copy(
                tail_hbm.at[idx_c], tails[b], gsems[b]
            ).start()

        def gather_wait(c, b):
            idx_c = idx_v.at[pl.ds(c * CHUNK, CHUNK)]
            pltpu.make_async_copy(
                table_hbm.at[idx_c, pl.ds(0, HEAD)],
                rows[b].at[:, pl.ds(0, HEAD)],
                gsems[b],
            ).wait()
            pltpu.make_async_copy(
                tail_hbm.at[idx_c], tails[b], gsems[b]
            ).wait()

        def out_slice(c):
            bi = batch0 + c // CPB
            h = c % CPB
            return out_hbm.at[bi, pl.ds(h * CHUNK, CHUNK)]

        def store(c, b):
            pltpu.make_async_copy(rows[b], out_slice(c), ssems[b]).start()

        def store_wait(c, b):
            pltpu.make_async_copy(rows[b], out_slice(c), ssems[b]).wait()

        def patch_tail(b):
            rb = rows[b]
            tb = tails[b]

            def row_body(r0, carry):
                for u in range(ROW_UNROLL):
                    r = r0 * ROW_UNROLL + u
                    x0 = tb[r, pl.ds(0, 16)]
                    x1 = tb[r, pl.ds(16, 16)]
                    x2 = tb[r, pl.ds(TAIL - 16, 16)]
                    rb[r, pl.ds(EMBED_DIM - 16, 16)] = x2
                    rb[r, pl.ds(HEAD + 16, 16)] = x1
                    rb[r, pl.ds(HEAD, 16)] = x0
                return carry

            lax.fori_loop(0, CHUNK // ROW_UNROLL, row_body, 0)

        gather(0, 0)
        gather(1, 1)

        def pair_body(i, carry):
            c0 = 2 * i
            c1 = 2 * i + 1
            gather_wait(c0, 0)
            patch_tail(0)
            store(c0, 0)
            gather_wait(c1, 1)
            patch_tail(1)
            store(c1, 1)

            @pl.when(i + 1 < N_CHUNKS // 2)
            def _():
                store_wait(c0, 0)
                gather(c0 + 2, 0)
                store_wait(c1, 1)
                gather(c1 + 2, 1)
            return carry

        lax.fori_loop(0, N_CHUNKS // 2, pair_body, 0)
        store_wait(N_CHUNKS - 2, 0)
        store_wait(N_CHUNKS - 1, 1)

    return k(table, tail_tab, idx_flat)


def kernel(inputs, table):
    tail_tab = jnp.pad(table[:, HEAD:], ((0, 0), (0, 128 - TAIL)))
    idx = inputs.reshape(-1)
    parts = [
        _sc_gather(table, tail_tab, idx[k * N_FLAT:(k + 1) * N_FLAT])
        for k in range(KSPLIT)
    ]
    return jnp.concatenate(parts, axis=0)


# speedup vs baseline: 1.5568x; 1.5568x over previous
import functools

import jax
import jax.numpy as jnp
from jax import lax
from jax.experimental import pallas as pl
from jax.experimental.pallas import tpu as pltpu
from jax.experimental.pallas import tpu_sc as plsc

NUM_TOKENS = 100000
EMBED_DIM = 300
HEAD = 256
TAIL = EMBED_DIM - HEAD
BATCH = 4096
SEQ = 200

NC = 2
NS = 16
NW = NC * NS

CHUNK = 64
N_FLAT = BATCH * SEQ
B_PER_W = N_FLAT // NW
BATCH_PER_W = BATCH // NW
N_CHUNKS = B_PER_W // CHUNK
ROW_UNROLL = 8


def _sc_gather(table, tail_tab, idx_flat):
    mesh = plsc.VectorSubcoreMesh(core_axis_name="c", subcore_axis_name="s")

    @functools.partial(
        pl.kernel,
        mesh=mesh,
        out_type=jax.ShapeDtypeStruct((N_FLAT, EMBED_DIM), jnp.float32),
        scratch_types=[
            pltpu.VMEM((B_PER_W,), jnp.int32),
            pltpu.VMEM((CHUNK, EMBED_DIM), jnp.float32),
            pltpu.VMEM((CHUNK, EMBED_DIM), jnp.float32),
            pltpu.VMEM((CHUNK, 128), jnp.float32),
            pltpu.VMEM((CHUNK, 128), jnp.float32),
            pltpu.SemaphoreType.DMA,
            pltpu.SemaphoreType.DMA,
            pltpu.SemaphoreType.DMA,
            pltpu.SemaphoreType.DMA,
        ],
        compiler_params=pltpu.CompilerParams(needs_layout_passes=False),
    )
    def k(table_hbm, tail_hbm, idx_hbm, out_hbm,
          idx_v, rows0, rows1, tail0, tail1, gs0, gs1, ss0, ss1):
        wid = lax.axis_index("s") * NC + lax.axis_index("c")
        base = wid * B_PER_W
        pltpu.sync_copy(idx_hbm.at[pl.ds(base, B_PER_W)], idx_v)

        rows = (rows0, rows1)
        tails = (tail0, tail1)
        gsems = (gs0, gs1)
        ssems = (ss0, ss1)

        def gather(c, b):
            idx_c = idx_v.at[pl.ds(c * CHUNK, CHUNK)]
            pltpu.make_async_copy(
                table_hbm.at[idx_c, pl.ds(0, HEAD)],
                rows[b].at[:, pl.ds(0, HEAD)],
                gsems[b],
            ).start()
            pltpu.make_async_copy(
                tail_hbm.at[idx_c], tails[b], gsems[b]
            ).start()

        def gather_wait(c, b):
            idx_c = idx_v.at[pl.ds(c * CHUNK, CHUNK)]
            pltpu.make_async_copy(
                table_hbm.at[idx_c, pl.ds(0, HEAD)],
                rows[b].at[:, pl.ds(0, HEAD)],
                gsems[b],
            ).wait()
            pltpu.make_async_copy(
                tail_hbm.at[idx_c], tails[b], gsems[b]
            ).wait()

        def out_slice(c):
            return out_hbm.at[pl.ds(base + c * CHUNK, CHUNK)]

        def store(c, b):
            pltpu.make_async_copy(rows[b], out_slice(c), ssems[b]).start()

        def store_wait(c, b):
            pltpu.make_async_copy(rows[b], out_slice(c), ssems[b]).wait()

        col_last = jnp.minimum(
            jnp.full((16,), HEAD + 32, jnp.int32) + lax.iota(jnp.int32, 16),
            jnp.full((16,), EMBED_DIM - 1, jnp.int32),
        )
        last_mask = lax.iota(jnp.int32, 16) < jnp.full((16,), TAIL - 32, jnp.int32)

        def patch_tail(b):
            rb = rows[b]
            tb = tails[b]

            def row_body(r0, carry):
                for u in range(ROW_UNROLL):
                    r = r0 * ROW_UNROLL + u
                    row16 = jnp.full((16,), 0, jnp.int32) + r
                    x0 = tb[r, pl.ds(0, 16)]
                    x1 = tb[r, pl.ds(16, 16)]
                    x2 = tb[r, pl.ds(32, 16)]
                    rb[r, pl.ds(HEAD, 16)] = x0
                    rb[r, pl.ds(HEAD + 16, 16)] = x1
                    plsc.store_scatter(rb, [row16, col_last], x2, mask=last_mask)
                return carry

            lax.fori_loop(0, CHUNK // ROW_UNROLL, row_body, 0)

        gather(0, 0)
        gather(1, 1)

        def pair_body(i, carry):
            c0 = 2 * i
            c1 = 2 * i + 1
            gather_wait(c0, 0)
            patch_tail(0)
            store(c0, 0)
            gather_wait(c1, 1)
            patch_tail(1)
            store(c1, 1)

            @pl.when(i + 1 < N_CHUNKS // 2)
            def _():
                store_wait(c0, 0)
                gather(c0 + 2, 0)
                store_wait(c1, 1)
                gather(c1 + 2, 1)
            return carry

        lax.fori_loop(0, N_CHUNKS // 2, pair_body, 0)
        store_wait(N_CHUNKS - 2, 0)
        store_wait(N_CHUNKS - 1, 1)

    return k(table, tail_tab, idx_flat)


def kernel(inputs, table):
    tail_tab = jnp.pad(table[:, HEAD:], ((0, 0), (0, 128 - TAIL)))
    out = _sc_gather(table, tail_tab, inputs.reshape(-1))
    return out.reshape(BATCH, SEQ, EMBED_DIM)
